# baseline (speedup 1.0000x reference)
import functools

import jax
import jax.numpy as jnp
from jax.experimental import pallas as pl
from jax.experimental.pallas import tpu as pltpu

K = 10
H = 128
DH = K * H
BLOCK_ROWS = 1000


def _cell_kernel(h_lo, h_hi, c_lo, c_hi, wf_ref, bf_ref, wiou_ref, biou_ref,
                 h_out_ref, c_out_ref, hcat_ref, w_bf_ref):
    @pl.when(pl.program_id(0) == 0)
    def _():
        w_bf_ref[:, 0:DH] = wf_ref[...].astype(jnp.bfloat16)
        w_bf_ref[:, DH:DH + 3 * H] = wiou_ref[...].astype(jnp.bfloat16)

    for k in range(K // 2):
        hcat_ref[:, k * H:(k + 1) * H] = h_lo[k].astype(jnp.bfloat16)
    for k in range(K // 2, K):
        hcat_ref[:, k * H:(k + 1) * H] = h_hi[k - K // 2].astype(jnp.bfloat16)
    h_cat = hcat_ref[...]
    res = jnp.dot(h_cat, w_bf_ref[...],
                  preferred_element_type=jnp.float32)
    f = jax.nn.sigmoid(res[:, 0:DH] + bf_ref[...])
    iou = res[:, DH:DH + 3 * H]
    c_red = f[:, 0:H] * c_lo[0]
    for k in range(1, K // 2):
        c_red += f[:, k * H:(k + 1) * H] * c_lo[k]
    for k in range(K // 2, K):
        c_red += f[:, k * H:(k + 1) * H] * c_hi[k - K // 2]
    iou += biou_ref[...]
    ig = jax.nn.sigmoid(iou[:, 0:H])
    og = jax.nn.sigmoid(iou[:, H:2 * H])
    ug = jnp.tanh(iou[:, 2 * H:3 * H])
    c_out = ig * ug + c_red
    c_out_ref[...] = c_out
    h_out_ref[...] = og * jnp.tanh(c_out)


@functools.partial(jax.jit, static_argnames=("interpret",))
def kernel(mailbox_h, mailbox_c, U_f_w, U_f_b, U_iou_w, b_iou,
           interpret=False):
    n = mailbox_h.shape[0]
    h_t = mailbox_h.transpose(1, 0, 2)
    c_t = mailbox_c.transpose(1, 0, 2)
    wf = U_f_w[:DH, :DH]
    wiou_t = U_iou_w[:, :DH].T
    bf = U_f_b[:DH].reshape(1, DH)
    grid = (pl.cdiv(n, BLOCK_ROWS),)
    h_out, c_out = pl.pallas_call(
        _cell_kernel,
        grid=grid,
        in_specs=[
            pl.BlockSpec((K // 2, BLOCK_ROWS, H), lambda i: (0, i, 0)),
            pl.BlockSpec((K // 2, BLOCK_ROWS, H), lambda i: (1, i, 0)),
            pl.BlockSpec((K // 2, BLOCK_ROWS, H), lambda i: (0, i, 0)),
            pl.BlockSpec((K // 2, BLOCK_ROWS, H), lambda i: (1, i, 0)),
            pl.BlockSpec((DH, DH), lambda i: (0, 0)),
            pl.BlockSpec((1, DH), lambda i: (0, 0)),
            pl.BlockSpec((DH, 3 * H), lambda i: (0, 0)),
            pl.BlockSpec((1, 3 * H), lambda i: (0, 0)),
        ],
        out_specs=[
            pl.BlockSpec((BLOCK_ROWS, H), lambda i: (i, 0)),
            pl.BlockSpec((BLOCK_ROWS, H), lambda i: (i, 0)),
        ],
        out_shape=[
            jax.ShapeDtypeStruct((n, H), jnp.float32),
            jax.ShapeDtypeStruct((n, H), jnp.float32),
        ],
        scratch_shapes=[
            pltpu.VMEM((BLOCK_ROWS, DH), jnp.bfloat16),
            pltpu.VMEM((DH, DH + 3 * H), jnp.bfloat16),
        ],
        compiler_params=pltpu.CompilerParams(
            dimension_semantics=("parallel",),
        ),
        interpret=interpret,
    )(h_t, h_t, c_t, c_t, wf, bf, wiou_t, b_iou)
    return (h_out, c_out)


# speedup vs baseline: 1.0056x; 1.0056x over previous
import functools

import jax
import jax.numpy as jnp
from jax.experimental import pallas as pl
from jax.experimental.pallas import tpu as pltpu

K = 10
H = 128
DH = K * H
BLOCK_ROWS = 1000


def _cell_kernel(h_ref, c_ref, wf_ref, bf_ref, wiou_ref, biou_ref,
                 h_out_ref, c_out_ref, hcat_ref, w_bf_ref):
    @pl.when(pl.program_id(0) == 0)
    def _():
        w_bf_ref[:, 0:DH] = wf_ref[...].astype(jnp.bfloat16)
        w_bf_ref[:, DH:DH + 3 * H] = wiou_ref[...].astype(jnp.bfloat16)

    for k in range(K):
        hcat_ref[:, k * H:(k + 1) * H] = h_ref[k].astype(jnp.bfloat16)
    h_cat = hcat_ref[...]
    res = jnp.dot(h_cat, w_bf_ref[...],
                  preferred_element_type=jnp.float32)
    f = jax.nn.sigmoid(res[:, 0:DH] + bf_ref[...])
    iou = res[:, DH:DH + 3 * H]
    c_red = f[:, 0:H] * c_ref[0]
    for k in range(1, K):
        c_red += f[:, k * H:(k + 1) * H] * c_ref[k]
    iou += biou_ref[...]
    ig = jax.nn.sigmoid(iou[:, 0:H])
    og = jax.nn.sigmoid(iou[:, H:2 * H])
    ug = jnp.tanh(iou[:, 2 * H:3 * H])
    c_out = ig * ug + c_red
    c_out_ref[...] = c_out
    h_out_ref[...] = og * jnp.tanh(c_out)


@functools.partial(jax.jit, static_argnames=("interpret",))
def kernel(mailbox_h, mailbox_c, U_f_w, U_f_b, U_iou_w, b_iou,
           interpret=False):
    n = mailbox_h.shape[0]
    h_t = mailbox_h.transpose(1, 0, 2)
    c_t = mailbox_c.transpose(1, 0, 2)
    wf = U_f_w[:DH, :DH]
    wiou_t = U_iou_w[:, :DH].T
    bf = U_f_b[:DH].reshape(1, DH)
    grid = (pl.cdiv(n, BLOCK_ROWS),)
    h_out, c_out = pl.pallas_call(
        _cell_kernel,
        grid=grid,
        in_specs=[
            pl.BlockSpec((K, BLOCK_ROWS, H), lambda i: (0, i, 0)),
            pl.BlockSpec((K, BLOCK_ROWS, H), lambda i: (0, i, 0)),
            pl.BlockSpec((DH, DH), lambda i: (0, 0)),
            pl.BlockSpec((1, DH), lambda i: (0, 0)),
            pl.BlockSpec((DH, 3 * H), lambda i: (0, 0)),
            pl.BlockSpec((1, 3 * H), lambda i: (0, 0)),
        ],
        out_specs=[
            pl.BlockSpec((BLOCK_ROWS, H), lambda i: (i, 0)),
            pl.BlockSpec((BLOCK_ROWS, H), lambda i: (i, 0)),
        ],
        out_shape=[
            jax.ShapeDtypeStruct((n, H), jnp.float32),
            jax.ShapeDtypeStruct((n, H), jnp.float32),
        ],
        scratch_shapes=[
            pltpu.VMEM((BLOCK_ROWS, DH), jnp.bfloat16),
            pltpu.VMEM((DH, DH + 3 * H), jnp.bfloat16),
        ],
        compiler_params=pltpu.CompilerParams(
            dimension_semantics=("parallel",),
        ),
        interpret=interpret,
    )(h_t, c_t, wf, bf, wiou_t, b_iou)
    return (h_out, c_out)
